# baseline (speedup 1.0000x reference)
import functools

import jax
import jax.numpy as jnp
from jax import lax
from jax.experimental import pallas as pl
from jax.experimental.pallas import tpu as pltpu
from jax.experimental.pallas import tpu_sc as plsc

EMB_DIM = 32
FRAG = 128
NBUF = 3
QROWS = 250112
QB = QROWS // 256
BATCH = 16384
HIST = 200


def _tc_transpose_body(t0, t1, t2, t3, out_ref):
    parts = [jnp.transpose(t[...], (1, 0)) for t in (t0, t1, t2, t3)]
    out_ref[...] = jnp.concatenate(parts, axis=1)


@functools.lru_cache(maxsize=None)
def _make_tc_transpose():
    def _imap(m, i):
        if m == 3:
            i = jnp.minimum(i, QB - 2)
        return (0, m * QB + i)

    specs = [pl.BlockSpec((32, 256), functools.partial(_imap, m))
             for m in range(4)]
    return pl.pallas_call(
        _tc_transpose_body,
        grid=(QB,),
        in_specs=specs,
        out_specs=pl.BlockSpec((256, 128), lambda i: (i, 0)),
        out_shape=jax.ShapeDtypeStruct((QROWS, 128), jnp.float32),
    )


@functools.lru_cache(maxsize=None)
def _make_sc_gather():
    info = plsc.get_sparse_core_info()
    nc, ns = info.num_cores, info.num_subcores
    nw = nc * ns
    c_per_w = (BATCH // FRAG) // nw
    frags = c_per_w * HIST
    assert frags >= NBUF + 2 and (frags - NBUF - 2) % NBUF == 0
    steady = (frags - NBUF - 2) // NBUF
    last = frags - 1

    mesh = plsc.VectorSubcoreMesh(core_axis_name="c", subcore_axis_name="s")

    @functools.partial(
        pl.kernel,
        out_type=jax.ShapeDtypeStruct((HIST, 4, BATCH // FRAG, 8, FRAG),
                                      jnp.float32),
        mesh=mesh,
        compiler_params=pltpu.CompilerParams(use_tc_tiling_on_sc=False,
                                             needs_layout_passes=False),
        scratch_types=[
            pltpu.VMEM((NBUF, FRAG), jnp.int32),
            pltpu.VMEM((NBUF, FRAG), jnp.int32),
            pltpu.VMEM((FRAG, EMB_DIM), jnp.float32),
            pltpu.VMEM((FRAG, EMB_DIM), jnp.float32),
            pltpu.VMEM((FRAG, EMB_DIM), jnp.float32),
            pltpu.VMEM((NBUF, EMB_DIM, FRAG), jnp.float32),
            pltpu.SemaphoreType.DMA((NBUF,)),
            pltpu.SemaphoreType.DMA((NBUF,)),
            pltpu.SemaphoreType.DMA((NBUF,)),
        ],
    )
    def gather_kernel(x4, t_rm, y5, idxv, ivv, gv0, gv1, gv2, tv,
                      isem, gsem, wsem):
        gvs = (gv0, gv1, gv2)
        wid = lax.axis_index("s") * nc + lax.axis_index("c")
        c0 = wid * c_per_w

        def frag_ct(f):
            return c0 + f % 4, f // 4

        def idx_copy(f, s):
            c, t = frag_ct(f)
            return pltpu.make_async_copy(
                x4.at[t // 8, c, t % 8], idxv.at[s], isem.at[s])

        def gath_copy(s):
            return pltpu.make_async_copy(
                t_rm.at[ivv.at[s]], gvs[s], gsem.at[s])

        def wb_copy(f, s, a):
            c, t = frag_ct(f)
            return pltpu.make_async_copy(
                tv.at[s, pl.ds(8 * a, 8)], y5.at[t, a, c], wsem.at[s])

        def remap(s):
            for q in range(FRAG // 16):
                v = idxv[s, pl.ds(16 * q, 16)]
                m = ((v >= QROWS).astype(jnp.int32)
                     + (v >= 2 * QROWS).astype(jnp.int32)
                     + (v >= 3 * QROWS).astype(jnp.int32))
                ivv[s, pl.ds(16 * q, 16)] = v * 4 - m * (4 * QROWS - 1)

        def transpose(s):
            iota = lax.iota(jnp.int32, 16)
            for j in range(EMB_DIM):
                cj = jnp.full((16,), j, jnp.int32)
                for q in range(FRAG // 16):
                    vals = plsc.load_gather(gvs[s], [iota + 16 * q, cj])
                    tv[s, j, pl.ds(16 * q, 16)] = vals

        def wb_start(f, s):
            for a in range(4):
                wb_copy(f, s, a).start()

        def wb_wait(f, s):
            for a in range(4):
                wb_copy(f, s, a).wait()

        idx_copy(0, 0).start()
        idx_copy(1, 1).start()
        idx_copy(0, 0).wait()
        remap(0)
        gath_copy(0).start()
        idx_copy(2, 2).start()
        idx_copy(1, 1).wait()
        remap(1)
        gath_copy(1).start()
        gath_copy(0).wait()
        transpose(0)
        wb_start(0, 0)
        idx_copy(3, 0).start()
        idx_copy(2, 2).wait()
        remap(2)
        gath_copy(2).start()
        gath_copy(1).wait()
        transpose(1)
        wb_start(1, 1)
        idx_copy(4, 1).start()

        def body(k, carry):
            for b in range(NBUF):
                f = NBUF + k * NBUF + b
                sp = (b + NBUF - 1) % NBUF
                idx_copy(f, b).wait()
                remap(b)
                wb_wait(f - NBUF, b)
                gath_copy(b).start()
                gath_copy(sp).wait()
                transpose(sp)
                wb_start(f - 1, sp)
                idx_copy(jnp.minimum(f + 2, last), sp).start()
            return carry

        lax.fori_loop(0, steady, body, 0)

        for f in (frags - 2, frags - 1):
            s = f % NBUF
            sp = (s + NBUF - 1) % NBUF
            idx_copy(f, s).wait()
            remap(s)
            wb_wait(f - NBUF, s)
            gath_copy(s).start()
            gath_copy(sp).wait()
            transpose(sp)
            wb_start(f - 1, sp)
            if f == frags - 2:
                idx_copy(last, sp).start()
        s = last % NBUF
        gath_copy(s).wait()
        transpose(s)
        wb_start(last, s)
        wb_wait(last - 2, (last - 2) % NBUF)
        wb_wait(last - 1, (last - 1) % NBUF)
        wb_wait(last, s)
        idx_copy(last, (last - 2) % NBUF).wait()

    return gather_kernel


def kernel(x, table):
    tt = table.T
    t128 = _make_tc_transpose()(tt, tt, tt, tt)
    t_rm = t128.reshape(4 * QROWS, EMB_DIM)
    x4 = (x.T.reshape(HIST // 8, 8, BATCH // FRAG, FRAG)
          .transpose(0, 2, 1, 3))
    y5 = _make_sc_gather()(x4, t_rm)
    return (y5.transpose(2, 4, 0, 1, 3)
            .reshape(BATCH, HIST, EMB_DIM))


# speedup vs baseline: 1.1557x; 1.1557x over previous
import functools

import jax
import jax.numpy as jnp
from jax import lax
from jax.experimental import pallas as pl
from jax.experimental.pallas import tpu as pltpu
from jax.experimental.pallas import tpu_sc as plsc

EMB_DIM = 32
IDX_ROW = 128
ROWS_PER_CHUNK = 8
CHUNK = IDX_ROW * ROWS_PER_CHUNK
NBUF = 3
QROWS = 250112
QB = QROWS // 256
BATCH = 16384
HIST = 200


def _tc_transpose_body(t0, t1, t2, t3, out_ref):
    parts = [jnp.transpose(t[...], (1, 0)) for t in (t0, t1, t2, t3)]
    out_ref[...] = jnp.concatenate(parts, axis=1)


@functools.lru_cache(maxsize=None)
def _make_tc_transpose():
    def _imap(m, i):
        if m == 3:
            i = jnp.minimum(i, QB - 2)
        return (0, m * QB + i)

    specs = [pl.BlockSpec((32, 256), functools.partial(_imap, m))
             for m in range(4)]
    return pl.pallas_call(
        _tc_transpose_body,
        grid=(QB,),
        in_specs=specs,
        out_specs=pl.BlockSpec((256, 128), lambda i: (i, 0)),
        out_shape=jax.ShapeDtypeStruct((QROWS, 128), jnp.float32),
    )


@functools.lru_cache(maxsize=None)
def _make_sc_gather(num_idx, vocab_pad):
    info = plsc.get_sparse_core_info()
    nc, ns = info.num_cores, info.num_subcores
    nw = nc * ns
    assert num_idx % (nw * CHUNK) == 0
    per_w = num_idx // nw
    rows_w = per_w // IDX_ROW
    chunks = per_w // CHUNK
    assert chunks >= NBUF + 1 and (chunks - NBUF - 1) % NBUF == 0

    mesh = plsc.VectorSubcoreMesh(core_axis_name="c", subcore_axis_name="s")

    @functools.partial(
        pl.kernel,
        out_type=jax.ShapeDtypeStruct((num_idx, EMB_DIM), jnp.float32),
        mesh=mesh,
        compiler_params=pltpu.CompilerParams(use_tc_tiling_on_sc=False),
        scratch_types=[
            pltpu.VMEM((NBUF, ROWS_PER_CHUNK, IDX_ROW), jnp.int32),
            pltpu.VMEM((NBUF, CHUNK, EMB_DIM), jnp.float32),
            pltpu.SemaphoreType.DMA((NBUF,)),
            pltpu.SemaphoreType.DMA((NBUF,)),
            pltpu.SemaphoreType.DMA((NBUF,)),
        ],
    )
    def gather_kernel(x_hbm, table_hbm, out_hbm, idx_v, rows_v,
                      isem, gsem, wsem):
        wid = lax.axis_index("s") * nc + lax.axis_index("c")
        row_base = wid * rows_w
        out_base = wid * per_w
        last = chunks - 1

        def idx_copy(g, s):
            return pltpu.make_async_copy(
                x_hbm.at[pl.ds(row_base + g * ROWS_PER_CHUNK, ROWS_PER_CHUNK)],
                idx_v.at[s], isem.at[s])

        def gath_copy(s, j):
            return pltpu.make_async_copy(
                table_hbm.at[idx_v.at[s, j]],
                rows_v.at[s, pl.ds(j * IDX_ROW, IDX_ROW)], gsem.at[s])

        def wb_copy(g, s):
            return pltpu.make_async_copy(
                rows_v.at[s], out_hbm.at[pl.ds(out_base + g * CHUNK, CHUNK)],
                wsem.at[s])

        def fire_gathers(s):
            for j in range(ROWS_PER_CHUNK):
                gath_copy(s, j).start()

        def drain_gathers(s):
            for j in range(ROWS_PER_CHUNK):
                gath_copy(s, j).wait()

        idx_copy(0, 0).start()
        idx_copy(1, 1).start()
        idx_copy(0, 0).wait()
        fire_gathers(0)
        idx_copy(2, 2).start()
        idx_copy(1, 1).wait()
        fire_gathers(1)
        drain_gathers(0)
        wb_copy(0, 0).start()
        idx_copy(3, 0).start()
        idx_copy(2, 2).wait()
        fire_gathers(2)
        drain_gathers(1)
        wb_copy(1, 1).start()
        idx_copy(4, 1).start()

        def body(k, carry):
            for b in range(NBUF):
                g = NBUF + k * NBUF + b
                sp = (b + NBUF - 1) % NBUF
                idx_copy(g, b).wait()
                wb_copy(g - NBUF, b).wait()
                fire_gathers(b)
                drain_gathers(sp)
                wb_copy(g - 1, sp).start()
                idx_copy(jnp.minimum(g + 2, last), sp).start()
            return carry

        lax.fori_loop(0, (chunks - NBUF - 1) // NBUF, body, 0)

        g = last
        s = last % NBUF
        sp = (s + NBUF - 1) % NBUF
        idx_copy(g, s).wait()
        wb_copy(g - NBUF, s).wait()
        fire_gathers(s)
        drain_gathers(sp)
        wb_copy(g - 1, sp).start()
        drain_gathers(s)
        wb_copy(g, s).start()
        wb_copy(g - 2, (s + 1) % NBUF).wait()
        wb_copy(g - 1, sp).wait()
        wb_copy(g, s).wait()
        idx_copy(last, (s + 1) % NBUF).wait()

    return gather_kernel


def kernel(x, table):
    tt = table.T
    t128 = _make_tc_transpose()(tt, tt, tt, tt)
    t_rm = t128.reshape(4 * QROWS, EMB_DIM)
    m = ((x >= QROWS).astype(jnp.int32)
         + (x >= 2 * QROWS).astype(jnp.int32)
         + (x >= 3 * QROWS).astype(jnp.int32))
    iv = x * 4 - m * (4 * QROWS - 1)
    iv_rows = (iv.T.reshape(HIST // 8, 8, BATCH // 128, 128)
               .transpose(0, 2, 1, 3)
               .reshape(BATCH * HIST // 128, 128))
    y = _make_sc_gather(BATCH * HIST, 4 * QROWS)(iv_rows, t_rm)
    return (y.reshape(HIST // 8, BATCH // 128, 8, 128, EMB_DIM)
            .transpose(1, 3, 0, 2, 4)
            .reshape(BATCH, HIST, EMB_DIM))


# speedup vs baseline: 1.1864x; 1.0266x over previous
import functools

import jax
import jax.numpy as jnp
from jax import lax
from jax.experimental import pallas as pl
from jax.experimental.pallas import tpu as pltpu
from jax.experimental.pallas import tpu_sc as plsc

EMB_DIM = 32
FRAG = 128
NBUF = 3
QROWS = 250112
QB = QROWS // 256
BATCH = 16384
HIST = 200


def _tc_transpose_body(t0, t1, t2, t3, out_ref):
    parts = [jnp.transpose(t[...], (1, 0)) for t in (t0, t1, t2, t3)]
    out_ref[...] = jnp.concatenate(parts, axis=1)


@functools.lru_cache(maxsize=None)
def _make_tc_transpose():
    def _imap(m, i):
        if m == 3:
            i = jnp.minimum(i, QB - 2)
        return (0, m * QB + i)

    specs = [pl.BlockSpec((32, 256), functools.partial(_imap, m))
             for m in range(4)]
    return pl.pallas_call(
        _tc_transpose_body,
        grid=(QB,),
        in_specs=specs,
        out_specs=pl.BlockSpec((256, 128), lambda i: (i, 0)),
        out_shape=jax.ShapeDtypeStruct((QROWS, 128), jnp.float32),
    )


@functools.lru_cache(maxsize=None)
def _make_sc_gather():
    info = plsc.get_sparse_core_info()
    nc, ns = info.num_cores, info.num_subcores
    nw = nc * ns
    c_per_w = (BATCH // FRAG) // nw
    frags = c_per_w * HIST
    assert frags >= NBUF + 2 and (frags - NBUF - 2) % NBUF == 0
    steady = (frags - NBUF - 2) // NBUF
    last = frags - 1

    mesh = plsc.VectorSubcoreMesh(core_axis_name="c", subcore_axis_name="s")

    @functools.partial(
        pl.kernel,
        out_type=jax.ShapeDtypeStruct((HIST, 4, BATCH // FRAG, 8, FRAG),
                                      jnp.float32),
        mesh=mesh,
        compiler_params=pltpu.CompilerParams(use_tc_tiling_on_sc=False,
                                             needs_layout_passes=False),
        scratch_types=[
            pltpu.VMEM((NBUF, FRAG), jnp.int32),
            pltpu.VMEM((NBUF, FRAG), jnp.int32),
            pltpu.VMEM((FRAG, EMB_DIM), jnp.float32),
            pltpu.VMEM((FRAG, EMB_DIM), jnp.float32),
            pltpu.VMEM((FRAG, EMB_DIM), jnp.float32),
            pltpu.VMEM((EMB_DIM, FRAG), jnp.float32),
            pltpu.VMEM((EMB_DIM, FRAG), jnp.float32),
            pltpu.VMEM((EMB_DIM, FRAG), jnp.float32),
            pltpu.SemaphoreType.DMA((NBUF,)),
            pltpu.SemaphoreType.DMA((NBUF,)),
            pltpu.SemaphoreType.DMA((NBUF,)),
        ],
    )
    def gather_kernel(x4, t_rm, y5, idxv, ivv, gv0, gv1, gv2,
                      tv0, tv1, tv2, isem, gsem, wsem):
        gvs = (gv0, gv1, gv2)
        tvs = (tv0, tv1, tv2)
        wid = lax.axis_index("s") * nc + lax.axis_index("c")
        c0 = wid * c_per_w

        def frag_ct(f):
            return c0 + f % 4, f // 4

        def idx_copy(f, s):
            c, t = frag_ct(f)
            return pltpu.make_async_copy(
                x4.at[t // 8, c, t % 8], idxv.at[s], isem.at[s])

        def gath_copy(s):
            return pltpu.make_async_copy(
                t_rm.at[ivv.at[s]], gvs[s], gsem.at[s])

        def wb_copy(f, s, a):
            c, t = frag_ct(f)
            return pltpu.make_async_copy(
                tvs[s].at[pl.ds(8 * a, 8)], y5.at[t, a, c], wsem.at[s])

        def remap(s):
            for q in range(FRAG // 16):
                v = idxv[s, pl.ds(16 * q, 16)]
                m = ((v >= QROWS).astype(jnp.int32)
                     + (v >= 2 * QROWS).astype(jnp.int32)
                     + (v >= 3 * QROWS).astype(jnp.int32))
                ivv[s, pl.ds(16 * q, 16)] = v * 4 - m * (4 * QROWS - 1)

        def transpose(s):
            iota = lax.iota(jnp.int32, 16)
            rh = [iota + 16 * h for h in range(EMB_DIM // 16)]
            for bp in range(FRAG):
                cb = jnp.full((16,), bp, jnp.int32)
                for h in range(EMB_DIM // 16):
                    v = gvs[s][bp, pl.ds(16 * h, 16)]
                    plsc.store_scatter(tvs[s], [rh[h], cb], v)

        def wb_start(f, s):
            for a in range(4):
                wb_copy(f, s, a).start()

        def wb_wait(f, s):
            for a in range(4):
                wb_copy(f, s, a).wait()

        idx_copy(0, 0).start()
        idx_copy(1, 1).start()
        idx_copy(0, 0).wait()
        remap(0)
        gath_copy(0).start()
        idx_copy(2, 2).start()
        idx_copy(1, 1).wait()
        remap(1)
        gath_copy(1).start()
        gath_copy(0).wait()
        transpose(0)
        wb_start(0, 0)
        idx_copy(3, 0).start()
        idx_copy(2, 2).wait()
        remap(2)
        gath_copy(2).start()
        gath_copy(1).wait()
        transpose(1)
        wb_start(1, 1)
        idx_copy(4, 1).start()

        def body(k, carry):
            for b in range(NBUF):
                f = NBUF + k * NBUF + b
                sp = (b + NBUF - 1) % NBUF
                idx_copy(f, b).wait()
                remap(b)
                wb_wait(f - NBUF, b)
                gath_copy(b).start()
                gath_copy(sp).wait()
                transpose(sp)
                wb_start(f - 1, sp)
                idx_copy(jnp.minimum(f + 2, last), sp).start()
            return carry

        lax.fori_loop(0, steady, body, 0)

        for f in (frags - 2, frags - 1):
            s = f % NBUF
            sp = (s + NBUF - 1) % NBUF
            idx_copy(f, s).wait()
            remap(s)
            wb_wait(f - NBUF, s)
            gath_copy(s).start()
            gath_copy(sp).wait()
            transpose(sp)
            wb_start(f - 1, sp)
            if f == frags - 2:
                idx_copy(last, sp).start()
        s = last % NBUF
        gath_copy(s).wait()
        transpose(s)
        wb_start(last, s)
        wb_wait(last - 2, (last - 2) % NBUF)
        wb_wait(last - 1, (last - 1) % NBUF)
        wb_wait(last, s)
        idx_copy(last, (last - 2) % NBUF).wait()

    return gather_kernel


def kernel(x, table):
    tt = table.T
    t128 = _make_tc_transpose()(tt, tt, tt, tt)
    t_rm = t128.reshape(4 * QROWS, EMB_DIM)
    x4 = (x.T.reshape(HIST // 8, 8, BATCH // FRAG, FRAG)
          .transpose(0, 2, 1, 3))
    y5 = _make_sc_gather()(x4, t_rm)
    return (y5.transpose(2, 4, 0, 1, 3)
            .reshape(BATCH, HIST, EMB_DIM))
